# baseline (speedup 1.0000x reference)
import jax
import jax.numpy as jnp
from jax import lax
from jax.experimental import pallas as pl
from jax.experimental.pallas import tpu as pltpu
from jax.experimental.pallas import tpu_sc as plsc

N_NODES = 50000
N_EDGES = 800000
D_IN = 100
HIDDEN = 256
NQ = 4
DQ = 25
DQP = 32
DOUT = NQ * DQP
N_SC = 2
N_TILES = 16
STRIPE = 3200
LAST_STRIPE = N_NODES - (N_TILES - 1) * STRIPE
EB = 96
BLK_PER_TILE = 528
NBLK = BLK_PER_TILE * N_TILES
E_PAD = NBLK * EB
ACC_ROWS = 50048
SUB = 4
NITER = (BLK_PER_TILE // SUB - 2) // 2
EB2 = 800
NBLK2 = N_EDGES // EB2
NW = N_SC * N_TILES
BLK2_PER_W = NBLK2 // NW
RB = 1000


def _stripe_copy(s, read, write):
    off = pl.multiple_of(s * STRIPE, STRIPE)

    @pl.when(s < N_TILES - 1)
    def _main():
        pltpu.sync_copy(read(pl.ds(off, STRIPE)), write(pl.ds(off, STRIPE)))

    @pl.when(s == N_TILES - 1)
    def _last():
        base = (N_TILES - 1) * STRIPE
        pltpu.sync_copy(read(pl.ds(base, LAST_STRIPE)),
                        write(pl.ds(base, LAST_STRIPE)))


def _agg_body(xq_hbm, edges_hbm, out_hbm, acc_sh, idx_v, rows_v, isems,
              gsems, ssems):
    c = lax.axis_index("c")
    s = lax.axis_index("s")

    for p in range(2):
        for cc in range(N_SC):
            q = 2 * cc + p

            @pl.when(c == cc)
            def _init(q=q):
                _stripe_copy(s, lambda d: xq_hbm.at[q, d],
                             lambda d: acc_sh.at[d])

        plsc.subcore_barrier()

        for cc in range(N_SC):
            q = 2 * cc + p

            @pl.when(c == cc)
            def _edges(q=q):
                table = xq_hbm.at[q]
                base = pl.multiple_of(s * BLK_PER_TILE, BLK_PER_TILE)

                def ld(b, j):
                    return pltpu.async_copy(edges_hbm.at[b], idx_v.at[j],
                                            isems.at[j])

                def ld_wait(b, j):
                    pltpu.make_async_copy(edges_hbm.at[b], idx_v.at[j],
                                          isems.at[j]).wait()

                def gat(j):
                    return pltpu.async_copy(table.at[idx_v.at[j, 0]],
                                            rows_v.at[j], gsems.at[j])

                def gat_wait(j):
                    pltpu.make_async_copy(table.at[idx_v.at[j, 0]],
                                          rows_v.at[j], gsems.at[j]).wait()

                def sca(j):
                    return pltpu.async_copy(rows_v.at[j],
                                            acc_sh.at[idx_v.at[j, 1]],
                                            ssems.at[j], add=True)

                def sca_wait(j):
                    pltpu.make_async_copy(rows_v.at[j],
                                          acc_sh.at[idx_v.at[j, 1]],
                                          ssems.at[j]).wait()

                for j in range(SUB):
                    ld(base + j, j)
                for j in range(SUB):
                    ld_wait(base + j, j)
                    gat(j)

                def body(i, carry):
                    boff = base + 2 * SUB * i
                    for j in range(SUB):
                        gat_wait(j)
                        sca(j)
                    for j in range(SUB):
                        @pl.when(i > 0)
                        def _(j=j):
                            sca_wait(SUB + j)
                        ld(boff + SUB + j, SUB + j)
                    for j in range(SUB):
                        ld_wait(boff + SUB + j, SUB + j)
                        gat(SUB + j)
                    for j in range(SUB):
                        gat_wait(SUB + j)
                        sca(SUB + j)
                    for j in range(SUB):
                        sca_wait(j)
                        ld(boff + 2 * SUB + j, j)
                    for j in range(SUB):
                        ld_wait(boff + 2 * SUB + j, j)
                        gat(j)
                    return carry

                lax.fori_loop(0, NITER, body, 0)

                eoff = base + 2 * SUB * NITER
                for j in range(SUB):
                    gat_wait(j)
                    sca(j)
                for j in range(SUB):
                    sca_wait(SUB + j)
                    ld(eoff + SUB + j, SUB + j)
                for j in range(SUB):
                    ld_wait(eoff + SUB + j, SUB + j)
                    gat(SUB + j)
                for j in range(SUB):
                    gat_wait(SUB + j)
                    sca(SUB + j)
                for j in range(SUB):
                    sca_wait(j)
                for j in range(SUB):
                    sca_wait(SUB + j)

        plsc.subcore_barrier()

        for cc in range(N_SC):
            q = 2 * cc + p

            @pl.when(c == cc)
            def _flush(q=q):
                _stripe_copy(s, lambda d: acc_sh.at[d],
                             lambda d: out_hbm.at[q, d])

        plsc.subcore_barrier()


_agg = pl.kernel(
    _agg_body,
    out_type=jax.ShapeDtypeStruct((NQ, N_NODES, DQP), jnp.float32),
    mesh=plsc.VectorSubcoreMesh(core_axis_name="c", subcore_axis_name="s"),
    scratch_types=[
        pltpu.VMEM_SHARED((ACC_ROWS, DQP), jnp.float32),
        pltpu.VMEM((2 * SUB, 2, EB), jnp.int32),
        pltpu.VMEM((2 * SUB, EB, DQP), jnp.float32),
        pltpu.SemaphoreType.DMA((2 * SUB,)),
        pltpu.SemaphoreType.DMA((2 * SUB,)),
        pltpu.SemaphoreType.DMA((2 * SUB,)),
    ],
    compiler_params=pltpu.CompilerParams(use_tc_tiling_on_sc=False),
)


def _mlp_body(h_ref, w1_ref, b1_ref, w2_ref, b2_ref, out_ref):
    hcat = jnp.concatenate([h_ref[q] for q in range(NQ)], axis=1)
    z = lax.dot_general(hcat, w1_ref[...], (((1,), (0,)), ((), ())),
                        preferred_element_type=jnp.float32,
                        precision=lax.Precision.HIGHEST)
    z = jnp.maximum(z + b1_ref[...], 0.0)
    w2s = jnp.sum(w2_ref[...], axis=1)
    out_ref[...] = (jnp.sum(z * w2s[None, :], axis=1, keepdims=True)
                    + jnp.sum(b2_ref[...]))


_mlp = pl.pallas_call(
    _mlp_body,
    grid=(N_NODES // RB,),
    in_specs=[
        pl.BlockSpec((NQ, RB, DQP), lambda i: (0, i, 0)),
        pl.BlockSpec((DOUT, HIDDEN), lambda i: (0, 0)),
        pl.BlockSpec((1, HIDDEN), lambda i: (0, 0)),
        pl.BlockSpec((HIDDEN, HIDDEN), lambda i: (0, 0)),
        pl.BlockSpec((1, HIDDEN), lambda i: (0, 0)),
    ],
    out_specs=pl.BlockSpec((RB, 1), lambda i: (i, 0)),
    out_shape=jax.ShapeDtypeStruct((N_NODES, 1), jnp.float32),
)


def _gather_body(s_hbm, dst_hbm, out_hbm, s_v, dst_v, out_v):
    c = lax.axis_index("c")
    s = lax.axis_index("s")
    w = s * N_SC + c
    pltpu.sync_copy(s_hbm, s_v)
    nb = BLK2_PER_W + jnp.where(w < NBLK2 % NW, 1, 0)

    def blk(i, carry):
        off = pl.multiple_of((w + i * NW) * EB2, EB2)
        pltpu.sync_copy(dst_hbm.at[pl.ds(off, EB2)], dst_v)

        def inner(j, c2):
            idx = dst_v[pl.ds(j * 16, 16)]
            out_v[pl.ds(j * 16, 16)] = plsc.load_gather(s_v, [idx])
            return c2

        lax.fori_loop(0, EB2 // 16, inner, 0)
        pltpu.sync_copy(out_v, out_hbm.at[pl.ds(off, EB2)])
        return carry

    lax.fori_loop(0, nb, blk, 0)


_gather = pl.kernel(
    _gather_body,
    out_type=jax.ShapeDtypeStruct((N_EDGES,), jnp.float32),
    mesh=plsc.VectorSubcoreMesh(core_axis_name="c", subcore_axis_name="s"),
    scratch_types=[
        pltpu.VMEM((N_NODES,), jnp.float32),
        pltpu.VMEM((EB2,), jnp.int32),
        pltpu.VMEM((EB2,), jnp.float32),
    ],
    compiler_params=pltpu.CompilerParams(needs_layout_passes=False),
)


def kernel(x, edge_index, pos_embeddings, W1, b1, W2, b2):
    perm = (jnp.arange(D_IN)[:, None]
            == (jnp.arange(NQ * DQP) % DQP
                + DQ * (jnp.arange(NQ * DQP) // DQP))[None, :]
            ).astype(jnp.float32)
    xf = x.reshape(N_NODES, D_IN)
    ei = edge_index.astype(jnp.int32)
    srcp = jnp.concatenate([ei[0], jnp.zeros((E_PAD - N_EDGES,), jnp.int32)])
    dstp = jnp.concatenate(
        [ei[1], jnp.full((E_PAD - N_EDGES,), N_NODES, jnp.int32)])
    edges = jnp.stack(
        [srcp.reshape(NBLK, EB), dstp.reshape(NBLK, EB)], axis=1)
    xp = lax.dot_general(xf, perm, (((1,), (0,)), ((), ())),
                         preferred_element_type=jnp.float32,
                         precision=lax.Precision.HIGHEST)
    xqs = xp.reshape(N_NODES, NQ, DQP).transpose(1, 0, 2)
    h4 = _agg(xqs, edges)
    W1p = jnp.pad(W1.reshape(NQ, DQ, HIDDEN),
                  ((0, 0), (0, DQP - DQ), (0, 0))).reshape(DOUT, HIDDEN)
    s = _mlp(h4, W1p, b1.reshape(1, HIDDEN), W2,
             b2.reshape(1, HIDDEN)).reshape(N_NODES)
    return _gather(s, ei[1])


# speedup vs baseline: 1.2907x; 1.2907x over previous
import jax
import jax.numpy as jnp
from jax import lax
from jax.experimental import pallas as pl
from jax.experimental.pallas import tpu as pltpu
from jax.experimental.pallas import tpu_sc as plsc

N_NODES = 50000
N_EDGES = 800000
D_IN = 100
HIDDEN = 256
NQ = 4
DQ = 25
DQP = 32
DOUT = NQ * DQP
N_SC = 2
N_TILES = 16
STRIPE = 3200
LAST_STRIPE = N_NODES - (N_TILES - 1) * STRIPE
EB = 128
BLK_PER_TILE = 392
NBLK = BLK_PER_TILE * N_TILES
E_PAD = NBLK * EB
ACC_ROWS = 50048
NGRP = BLK_PER_TILE // 4
EB2 = 800
NBLK2 = N_EDGES // EB2
NW = N_SC * N_TILES
BLK2_PER_W = NBLK2 // NW
RB = 1000


def _stripe_copy(s, read, write):
    off = pl.multiple_of(s * STRIPE, STRIPE)

    @pl.when(s < N_TILES - 1)
    def _main():
        pltpu.sync_copy(read(pl.ds(off, STRIPE)), write(pl.ds(off, STRIPE)))

    @pl.when(s == N_TILES - 1)
    def _last():
        base = (N_TILES - 1) * STRIPE
        pltpu.sync_copy(read(pl.ds(base, LAST_STRIPE)),
                        write(pl.ds(base, LAST_STRIPE)))


def _agg_body(xq_hbm, edges_hbm, out_hbm, acc_sh, idx_a, idx_b, rows_a,
              rows_b, isem_a, isem_b, gsem_a, gsem_b):
    c = lax.axis_index("c")
    s = lax.axis_index("s")

    for p in range(2):
        for cc in range(N_SC):
            q = 2 * cc + p

            @pl.when(c == cc)
            def _init(q=q):
                _stripe_copy(s, lambda d: xq_hbm.at[q, d],
                             lambda d: acc_sh.at[d])

        plsc.subcore_barrier()

        for cc in range(N_SC):
            q = 2 * cc + p

            @pl.when(c == cc)
            def _edges(q=q):
                table = xq_hbm.at[q]
                base = pl.multiple_of(s * BLK_PER_TILE, BLK_PER_TILE)

                def idx_slice(off):
                    return edges_hbm.at[pl.ds(pl.multiple_of(off, 2), 2)]

                def gather(j, idx, rows, sem):
                    return pltpu.async_copy(table.at[idx.at[j, 0]],
                                            rows.at[j], sem)

                def gather_wait(j, idx, rows, sem):
                    pltpu.make_async_copy(table.at[idx.at[j, 0]],
                                          rows.at[j], sem).wait()

                def scat(j, idx, rows):
                    pltpu.sync_copy(rows.at[j], acc_sh.at[idx.at[j, 1]],
                                    add=True)

                pltpu.async_copy(idx_slice(base), idx_a, isem_a)
                pltpu.async_copy(idx_slice(base + 2), idx_b, isem_b)
                pltpu.make_async_copy(idx_slice(base), idx_a, isem_a).wait()
                gather(0, idx_a, rows_a, gsem_a)
                gather(1, idx_a, rows_a, gsem_a)

                def grp(g, carry):
                    koff = pl.multiple_of(base + 4 * g, 2)
                    gather_wait(0, idx_a, rows_a, gsem_a)
                    gather_wait(1, idx_a, rows_a, gsem_a)
                    pltpu.make_async_copy(idx_slice(koff + 2), idx_b,
                                          isem_b).wait()
                    hb0 = gather(0, idx_b, rows_b, gsem_b)
                    hb1 = gather(1, idx_b, rows_b, gsem_b)
                    scat(0, idx_a, rows_a)
                    scat(1, idx_a, rows_a)
                    hla = pltpu.async_copy(idx_slice(koff + 4), idx_a, isem_a)
                    hb0.wait()
                    hb1.wait()
                    hla.wait()
                    gather(0, idx_a, rows_a, gsem_a)
                    gather(1, idx_a, rows_a, gsem_a)
                    scat(0, idx_b, rows_b)
                    scat(1, idx_b, rows_b)
                    pltpu.async_copy(idx_slice(koff + 6), idx_b, isem_b)
                    return carry

                lax.fori_loop(0, NGRP - 1, grp, 0)

                gather_wait(0, idx_a, rows_a, gsem_a)
                gather_wait(1, idx_a, rows_a, gsem_a)
                pltpu.make_async_copy(idx_slice(base + BLK_PER_TILE - 2),
                                      idx_b, isem_b).wait()
                hb0 = gather(0, idx_b, rows_b, gsem_b)
                hb1 = gather(1, idx_b, rows_b, gsem_b)
                scat(0, idx_a, rows_a)
                scat(1, idx_a, rows_a)
                hb0.wait()
                hb1.wait()
                scat(0, idx_b, rows_b)
                scat(1, idx_b, rows_b)

        plsc.subcore_barrier()

        for cc in range(N_SC):
            q = 2 * cc + p

            @pl.when(c == cc)
            def _flush(q=q):
                _stripe_copy(s, lambda d: acc_sh.at[d],
                             lambda d: out_hbm.at[q, d])

        plsc.subcore_barrier()


_agg = pl.kernel(
    _agg_body,
    out_type=jax.ShapeDtypeStruct((NQ, N_NODES, DQP), jnp.float32),
    mesh=plsc.VectorSubcoreMesh(core_axis_name="c", subcore_axis_name="s"),
    scratch_types=[
        pltpu.VMEM_SHARED((ACC_ROWS, DQP), jnp.float32),
        pltpu.VMEM((2, 2, EB), jnp.int32),
        pltpu.VMEM((2, 2, EB), jnp.int32),
        pltpu.VMEM((2, EB, DQP), jnp.float32),
        pltpu.VMEM((2, EB, DQP), jnp.float32),
        pltpu.SemaphoreType.DMA,
        pltpu.SemaphoreType.DMA,
        pltpu.SemaphoreType.DMA,
        pltpu.SemaphoreType.DMA,
    ],
    compiler_params=pltpu.CompilerParams(use_tc_tiling_on_sc=False),
)


def _mlp_body(h_ref, w1_ref, b1_ref, w2_ref, b2_ref, out_ref):
    hcat = jnp.concatenate([h_ref[q] for q in range(NQ)], axis=1)
    z = lax.dot_general(hcat, w1_ref[...], (((1,), (0,)), ((), ())),
                        preferred_element_type=jnp.float32,
                        precision=lax.Precision.HIGHEST)
    z = jnp.maximum(z + b1_ref[...], 0.0)
    w2s = jnp.sum(w2_ref[...], axis=1)
    out_ref[...] = (jnp.sum(z * w2s[None, :], axis=1, keepdims=True)
                    + jnp.sum(b2_ref[...]))


_mlp = pl.pallas_call(
    _mlp_body,
    grid=(N_NODES // RB,),
    in_specs=[
        pl.BlockSpec((NQ, RB, DQP), lambda i: (0, i, 0)),
        pl.BlockSpec((DOUT, HIDDEN), lambda i: (0, 0)),
        pl.BlockSpec((1, HIDDEN), lambda i: (0, 0)),
        pl.BlockSpec((HIDDEN, HIDDEN), lambda i: (0, 0)),
        pl.BlockSpec((1, HIDDEN), lambda i: (0, 0)),
    ],
    out_specs=pl.BlockSpec((RB, 1), lambda i: (i, 0)),
    out_shape=jax.ShapeDtypeStruct((N_NODES, 1), jnp.float32),
)


def _gather_body(s_hbm, dst_hbm, out_hbm, s_v, dst_v, out_v):
    c = lax.axis_index("c")
    s = lax.axis_index("s")
    w = s * N_SC + c
    pltpu.sync_copy(s_hbm, s_v)
    nb = BLK2_PER_W + jnp.where(w < NBLK2 % NW, 1, 0)

    def blk(i, carry):
        off = pl.multiple_of((w + i * NW) * EB2, EB2)
        pltpu.sync_copy(dst_hbm.at[pl.ds(off, EB2)], dst_v)

        def inner(j, c2):
            idx = dst_v[pl.ds(j * 16, 16)]
            out_v[pl.ds(j * 16, 16)] = plsc.load_gather(s_v, [idx])
            return c2

        lax.fori_loop(0, EB2 // 16, inner, 0)
        pltpu.sync_copy(out_v, out_hbm.at[pl.ds(off, EB2)])
        return carry

    lax.fori_loop(0, nb, blk, 0)


_gather = pl.kernel(
    _gather_body,
    out_type=jax.ShapeDtypeStruct((N_EDGES,), jnp.float32),
    mesh=plsc.VectorSubcoreMesh(core_axis_name="c", subcore_axis_name="s"),
    scratch_types=[
        pltpu.VMEM((N_NODES,), jnp.float32),
        pltpu.VMEM((EB2,), jnp.int32),
        pltpu.VMEM((EB2,), jnp.float32),
    ],
    compiler_params=pltpu.CompilerParams(needs_layout_passes=False),
)


def kernel(x, edge_index, pos_embeddings, W1, b1, W2, b2):
    perm = (jnp.arange(D_IN)[:, None]
            == (jnp.arange(NQ * DQP) % DQP
                + DQ * (jnp.arange(NQ * DQP) // DQP))[None, :]
            ).astype(jnp.float32)
    xf = x.reshape(N_NODES, D_IN)
    ei = edge_index.astype(jnp.int32)
    srcp = jnp.concatenate([ei[0], jnp.zeros((E_PAD - N_EDGES,), jnp.int32)])
    dstp = jnp.concatenate(
        [ei[1], jnp.full((E_PAD - N_EDGES,), N_NODES, jnp.int32)])
    edges = jnp.stack(
        [srcp.reshape(NBLK, EB), dstp.reshape(NBLK, EB)], axis=1)
    xp = lax.dot_general(xf, perm, (((1,), (0,)), ((), ())),
                         preferred_element_type=jnp.float32,
                         precision=lax.Precision.HIGHEST)
    xqs = xp.reshape(N_NODES, NQ, DQP).transpose(1, 0, 2)
    h4 = _agg(xqs, edges)
    W1p = jnp.pad(W1.reshape(NQ, DQ, HIDDEN),
                  ((0, 0), (0, DQP - DQ), (0, 0))).reshape(DOUT, HIDDEN)
    s = _mlp(h4, W1p, b1.reshape(1, HIDDEN), W2,
             b2.reshape(1, HIDDEN)).reshape(N_NODES)
    return _gather(s, ei[1])
